# baseline (speedup 1.0000x reference)
import jax
import jax.numpy as jnp
from jax.experimental import pallas as pl

P = 16
C_MAP = 96
D_STRUC = 64
D_SEG = 32
FH = 32
FW = 32
NN = FH * FW
IG = 4
RG = FH // IG
YB = RG * P


def _embed_body(img_ref, wp_ref, bm_ref, cf_ref):
    v = img_ref[0]
    v5 = v.reshape(3, RG, P, FW, P)
    acc = jnp.zeros((RG * FW, C_MAP), dtype=jnp.float32)
    for c in range(3):
        for py in range(P):
            xs = v5[c, :, py].reshape(RG * FW, P)
            acc = acc + jnp.dot(xs, wp_ref[c * P + py],
                                preferred_element_type=jnp.float32)
    cf_ref[0] = acc + bm_ref[:]


def _graph_body(cf_ref, wcs_ref, wstruc_ref, wseg_ref, wposT_ref,
                pred_ref, loss_ref, acc_ref):
    b = pl.program_id(0)
    num_images = pl.num_programs(0)

    cf = cf_ref[0]

    z32 = jnp.zeros((FW, C_MAP), dtype=jnp.float32)
    z1 = jnp.zeros((1, C_MAP), dtype=jnp.float32)
    up = jnp.concatenate([z32, cf[:-FW]], axis=0)
    dn = jnp.concatenate([cf[FW:], z32], axis=0)
    lf = jnp.concatenate([z1, cf[:-1]], axis=0)
    rt = jnp.concatenate([cf[1:], z1], axis=0)
    n_idx = jax.lax.broadcasted_iota(jnp.int32, (NN, 1), 0)
    jj = n_idx % FW
    agg = (up + dn
           + jnp.where(jj != 0, lf, 0.0)
           + jnp.where(jj != FW - 1, rt, 0.0))

    x = (jj.astype(jnp.float32) + 0.5) / FW
    y = ((n_idx // FW).astype(jnp.float32) + 0.5) / FH

    struc = jnp.dot(agg, wcs_ref[:], preferred_element_type=jnp.float32)
    struc = struc + x * wstruc_ref[0:1, :] + y * wstruc_ref[1:2, :]
    struc = jnp.maximum(struc, 0.0)

    ssf = jnp.dot(struc, wseg_ref[:], preferred_element_type=jnp.float32)

    k_idx = jax.lax.broadcasted_iota(jnp.int32, (1, NN), 1)
    xk = (jnp.astype(k_idx % FW, jnp.float32) + 0.5) / FW
    yk = (jnp.astype(k_idx // FW, jnp.float32) + 0.5) / FH
    posT = wposT_ref[:, 0:1] * xk + wposT_ref[:, 1:2] * yk

    pred = jnp.dot(ssf, posT, preferred_element_type=jnp.float32)
    pred_ref[:] = pred.reshape(NN, FH, FW)
    loss_acc = jnp.sum(jnp.maximum(pred, 0.0)
                       + jnp.log1p(jnp.exp(-jnp.abs(pred))))
    neg_acc = jnp.sum(jnp.where(pred < 0.0, 1.0, 0.0))

    pos = x * wposT_ref[:, 0:1].reshape(1, D_SEG) + y * wposT_ref[:, 1:2].reshape(1, D_SEG)
    diag = jnp.sum(ssf * pos, axis=1, keepdims=True)
    loss_part = loss_acc - jnp.sum(diag)
    correct = neg_acc + jnp.sum(jnp.where(diag >= 0.0, 1.0, 0.0)) \
        - jnp.sum(jnp.where(diag < 0.0, 1.0, 0.0))

    prev_loss = jnp.where(b == 0, 0.0, loss_ref[0:1, 0:1])
    prev_cnt = jnp.where(b == 0, 0.0, acc_ref[0:1, 0:1])
    tot_loss = prev_loss + loss_part
    tot_cnt = prev_cnt + correct
    loss_ref[0:1, 0:1] = tot_loss
    acc_ref[0:1, 0:1] = jnp.where(
        b == num_images - 1,
        100.0 * tot_cnt / (num_images * NN * NN),
        tot_cnt)


@jax.jit
def kernel(images, W_map, b_map, W_con, W_cs, W_struc, W_seg, W_pos):
    del W_con
    B = images.shape[0]
    N = B * NN
    wp = (W_map.reshape(C_MAP, 3, P, P).transpose(1, 2, 3, 0)
          .reshape(3 * P, P, C_MAP))
    bm = b_map.reshape(1, C_MAP)
    WposT = W_pos.T

    cf = pl.pallas_call(
        _embed_body,
        grid=(B, IG),
        in_specs=[
            pl.BlockSpec((1, 3, YB, FW * P), lambda b, g: (b, 0, g, 0)),
            pl.BlockSpec((3 * P, P, C_MAP), lambda b, g: (0, 0, 0)),
            pl.BlockSpec((1, C_MAP), lambda b, g: (0, 0)),
        ],
        out_specs=pl.BlockSpec((1, RG * FW, C_MAP), lambda b, g: (b, g, 0)),
        out_shape=jax.ShapeDtypeStruct((B, NN, C_MAP), jnp.float32),
    )(images, wp, bm)

    pred, loss, acc = pl.pallas_call(
        _graph_body,
        grid=(B,),
        in_specs=[
            pl.BlockSpec((1, NN, C_MAP), lambda b: (b, 0, 0)),
            pl.BlockSpec((C_MAP, D_STRUC), lambda b: (0, 0)),
            pl.BlockSpec((2, D_STRUC), lambda b: (0, 0)),
            pl.BlockSpec((D_STRUC, D_SEG), lambda b: (0, 0)),
            pl.BlockSpec((D_SEG, 2), lambda b: (0, 0)),
        ],
        out_specs=[
            pl.BlockSpec((NN, FH, FW), lambda b: (b, 0, 0)),
            pl.BlockSpec((1, 1), lambda b: (0, 0)),
            pl.BlockSpec((1, 1), lambda b: (0, 0)),
        ],
        out_shape=[
            jax.ShapeDtypeStruct((N, FH, FW), jnp.float32),
            jax.ShapeDtypeStruct((1, 1), jnp.float32),
            jax.ShapeDtypeStruct((1, 1), jnp.float32),
        ],
    )(cf, W_cs, W_struc, W_seg, WposT)

    return pred, loss.reshape(()), acc.reshape(())


# speedup vs baseline: 2.5258x; 2.5258x over previous
import jax
import jax.numpy as jnp
from jax.experimental import pallas as pl

P = 16
C_MAP = 96
D_STRUC = 64
D_SEG = 32
FH = 32
FW = 32
NN = FH * FW


def _body(img_ref, wp_ref, bm_ref, wcsT_ref, wstrucT_ref, wsegT_ref,
          wposT_ref, pred_ref, loss_ref, acc_ref):
    b = pl.program_id(0)
    num_images = pl.num_programs(0)

    v = img_ref[0]
    v5 = v.reshape(3, FH, P, FW, P)
    acc = jnp.zeros((NN, C_MAP), dtype=jnp.float32)
    for c in range(3):
        for py in range(P):
            xs = v5[c, :, py].reshape(NN, P)
            acc = acc + jnp.dot(xs, wp_ref[c * P + py],
                                preferred_element_type=jnp.float32)
    cfT = (acc + bm_ref[:]).T

    z32 = jnp.zeros((C_MAP, FW), dtype=jnp.float32)
    z1 = jnp.zeros((C_MAP, 1), dtype=jnp.float32)
    up = jnp.concatenate([z32, cfT[:, :-FW]], axis=1)
    dn = jnp.concatenate([cfT[:, FW:], z32], axis=1)
    lf = jnp.concatenate([z1, cfT[:, :-1]], axis=1)
    rt = jnp.concatenate([cfT[:, 1:], z1], axis=1)
    n_idx = jax.lax.broadcasted_iota(jnp.int32, (1, NN), 1)
    jj = n_idx % FW
    aggT = (up + dn
            + jnp.where(jj != 0, lf, 0.0)
            + jnp.where(jj != FW - 1, rt, 0.0))

    xn = (jj.astype(jnp.float32) + 0.5) / FW
    yn = ((n_idx // FW).astype(jnp.float32) + 0.5) / FH

    strucT = jnp.dot(wcsT_ref[:], aggT, preferred_element_type=jnp.float32)
    strucT = strucT + wstrucT_ref[:, 0:1] * xn + wstrucT_ref[:, 1:2] * yn
    strucT = jnp.maximum(strucT, 0.0)

    ssfT = jnp.dot(wsegT_ref[:], strucT, preferred_element_type=jnp.float32)

    posKn = wposT_ref[:, 0:1] * xn + wposT_ref[:, 1:2] * yn
    k_idx = jax.lax.broadcasted_iota(jnp.int32, (NN, 1), 0)
    xk = (jnp.astype(k_idx % FW, jnp.float32) + 0.5) / FW
    yk = (jnp.astype(k_idx // FW, jnp.float32) + 0.5) / FH
    posK = xk * wposT_ref[:, 0:1].reshape(1, D_SEG) \
        + yk * wposT_ref[:, 1:2].reshape(1, D_SEG)

    predT = jnp.dot(posK, ssfT, preferred_element_type=jnp.float32)
    pred_ref[:] = predT.reshape(FH, FW, NN)

    diag = jnp.sum(posKn * ssfT, axis=0, keepdims=True)
    loss_elem = jnp.sum(jnp.maximum(predT, 0.0)
                        + jnp.log1p(jnp.exp(-jnp.abs(predT))))
    neg_all = jnp.sum(jnp.where(predT < 0.0, 1.0, 0.0))
    loss_part = loss_elem - jnp.sum(diag)
    correct = neg_all + jnp.sum(jnp.where(diag >= 0.0, 1.0, 0.0)) \
        - jnp.sum(jnp.where(diag < 0.0, 1.0, 0.0))

    prev_loss = jnp.where(b == 0, 0.0, loss_ref[0:1, 0:1])
    prev_cnt = jnp.where(b == 0, 0.0, acc_ref[0:1, 0:1])
    tot_loss = prev_loss + loss_part
    tot_cnt = prev_cnt + correct
    loss_ref[0:1, 0:1] = tot_loss
    acc_ref[0:1, 0:1] = jnp.where(
        b == num_images - 1,
        100.0 * tot_cnt / (num_images * NN * NN),
        tot_cnt)


@jax.jit
def kernel(images, W_map, b_map, W_con, W_cs, W_struc, W_seg, W_pos):
    del W_con
    B = images.shape[0]
    N = B * NN
    wp = (W_map.reshape(C_MAP, 3, P, P).transpose(1, 2, 3, 0)
          .reshape(3 * P, P, C_MAP))
    bm = b_map.reshape(1, C_MAP)
    WposT = W_pos.T

    predT, loss, acc = pl.pallas_call(
        _body,
        grid=(B,),
        in_specs=[
            pl.BlockSpec((1, 3, FH * P, FW * P), lambda b: (b, 0, 0, 0)),
            pl.BlockSpec((3 * P, P, C_MAP), lambda b: (0, 0, 0)),
            pl.BlockSpec((1, C_MAP), lambda b: (0, 0)),
            pl.BlockSpec((D_STRUC, C_MAP), lambda b: (0, 0)),
            pl.BlockSpec((D_STRUC, 2), lambda b: (0, 0)),
            pl.BlockSpec((D_SEG, D_STRUC), lambda b: (0, 0)),
            pl.BlockSpec((D_SEG, 2), lambda b: (0, 0)),
        ],
        out_specs=[
            pl.BlockSpec((FH, FW, NN), lambda b: (0, 0, b)),
            pl.BlockSpec((1, 1), lambda b: (0, 0)),
            pl.BlockSpec((1, 1), lambda b: (0, 0)),
        ],
        out_shape=[
            jax.ShapeDtypeStruct((FH, FW, N), jnp.float32),
            jax.ShapeDtypeStruct((1, 1), jnp.float32),
            jax.ShapeDtypeStruct((1, 1), jnp.float32),
        ],
    )(images, wp, bm, W_cs.T, W_struc.T, W_seg.T, WposT)

    pred_maps = jnp.transpose(predT, (2, 0, 1))
    return pred_maps, loss.reshape(()), acc.reshape(())
